# baseline (speedup 1.0000x reference)
import jax
import jax.numpy as jnp
from jax.experimental import pallas as pl
from jax.experimental.pallas import tpu as pltpu

_THRESHOLD = 0.1
_BLK_ROWS = 256


def _body(x_ref, o_ref):
    j = pl.program_id(1)
    nj = pl.num_programs(1)

    @pl.when(j < nj // 2)
    def _copy():
        o_ref[...] = x_ref[...]

    @pl.when(j >= nj // 2)
    def _mask():
        v = x_ref[...]
        o_ref[...] = jnp.where(jnp.abs(v) > _THRESHOLD, v, 0.0)


def kernel(x):
    b, s, d = x.shape
    grid = (b, s // _BLK_ROWS)
    return pl.pallas_call(
        _body,
        grid=grid,
        in_specs=[pl.BlockSpec((1, _BLK_ROWS, d), lambda i, j: (i, j, 0))],
        out_specs=pl.BlockSpec((1, _BLK_ROWS, d), lambda i, j: (i, j, 0)),
        out_shape=jax.ShapeDtypeStruct(x.shape, x.dtype),
    )(x)


# speedup vs baseline: 1.0123x; 1.0123x over previous
import jax
import jax.numpy as jnp
from jax.experimental import pallas as pl
from jax.experimental.pallas import tpu as pltpu

_THRESHOLD = 0.1
_BLK_ROWS = 512


def _body(x_ref, o_ref):
    j = pl.program_id(1)
    nj = pl.num_programs(1)

    @pl.when(j < nj // 2)
    def _copy():
        o_ref[...] = x_ref[...]

    @pl.when(j >= nj // 2)
    def _mask():
        v = x_ref[...]
        o_ref[...] = jnp.where(jnp.abs(v) > _THRESHOLD, v, 0.0)


def kernel(x):
    b, s, d = x.shape
    grid = (b, s // _BLK_ROWS)
    return pl.pallas_call(
        _body,
        grid=grid,
        in_specs=[pl.BlockSpec((1, _BLK_ROWS, d), lambda i, j: (i, j, 0))],
        out_specs=pl.BlockSpec((1, _BLK_ROWS, d), lambda i, j: (i, j, 0)),
        out_shape=jax.ShapeDtypeStruct(x.shape, x.dtype),
    )(x)
